# baseline (speedup 1.0000x reference)
import functools

import jax
import jax.numpy as jnp
from jax import lax
from jax.experimental import pallas as pl
from jax.experimental.pallas import tpu as pltpu
from jax.experimental.pallas import tpu_sc as plsc

D = 256
K = 8192
N = 16 * 576

BN = 768
BK = 2048
NN = N // BN
NK = K // BK


def _argmin_body(x_ref, cb_ref, out_ref, best_val, best_idx):
    k = pl.program_id(1)

    @pl.when(k == 0)
    def _init():
        best_val[...] = jnp.full_like(best_val, jnp.inf)
        best_idx[...] = jnp.zeros_like(best_idx)

    x = x_ref[...]
    cb = cb_ref[...]
    a2 = jnp.sum(x * x, axis=1, keepdims=True)
    b2 = jnp.sum(cb * cb, axis=1)[None, :]
    s = lax.dot_general(x, cb, (((1,), (1,)), ((), ())),
                        preferred_element_type=jnp.float32)
    d = jnp.sqrt(jnp.maximum(a2 + b2 - 2.0 * s, 0.0))

    m = jnp.min(d, axis=1, keepdims=True)
    cols = lax.broadcasted_iota(jnp.int32, d.shape, 1)
    li = jnp.min(jnp.where(d == m, cols, jnp.int32(2**30)),
                 axis=1, keepdims=True)
    gi = li + k * BK

    upd = m < best_val[...]
    best_idx[...] = jnp.where(upd, gi, best_idx[...])
    best_val[...] = jnp.where(upd, m, best_val[...])

    @pl.when(k == pl.num_programs(1) - 1)
    def _flush():
        out_ref[0, 0, :] = best_idx[...][:, 0]


_argmin_call = pl.pallas_call(
    _argmin_body,
    grid=(NN, NK),
    in_specs=[
        pl.BlockSpec((BN, D), lambda n, k: (n, 0)),
        pl.BlockSpec((BK, D), lambda n, k: (k, 0)),
    ],
    out_specs=pl.BlockSpec((1, 1, BN), lambda n, k: (n, 0, 0)),
    out_shape=jax.ShapeDtypeStruct((NN, 1, BN), jnp.int32),
    scratch_shapes=[
        pltpu.VMEM((BN, 1), jnp.float32),
        pltpu.VMEM((BN, 1), jnp.int32),
    ],
    compiler_params=pltpu.CompilerParams(
        dimension_semantics=("parallel", "arbitrary"),
    ),
)


_NW = 32
_BPW = N // _NW


def _gather_body(table_hbm, idx_hbm, out_hbm, idx_v, rows_v, sem):
    wid = lax.axis_index("s") * 2 + lax.axis_index("c")
    base = wid * _BPW
    pltpu.sync_copy(idx_hbm.at[pl.ds(base, _BPW)], idx_v)
    pltpu.async_copy(table_hbm.at[idx_v], rows_v, sem).wait()
    pltpu.sync_copy(rows_v, out_hbm.at[pl.ds(base, _BPW)])


@functools.cache
def _gather_call():
    return functools.partial(
        pl.kernel,
        out_type=jax.ShapeDtypeStruct((N, D), jnp.float32),
        mesh=plsc.VectorSubcoreMesh(core_axis_name="c", subcore_axis_name="s"),
        scratch_types=[
            pltpu.VMEM((_BPW,), jnp.int32),
            pltpu.VMEM((_BPW, D), jnp.float32),
            pltpu.SemaphoreType.DMA,
        ],
    )(_gather_body)


def kernel(x, codebook):
    xf = x.reshape(N, D)
    idx_blocks = _argmin_call(xf, codebook)
    indices = idx_blocks.reshape(N)
    codes = _gather_call()(codebook, indices)
    idx_shape = list(x.shape)
    idx_shape[-1] = 1
    return codes.reshape(x.shape), indices.reshape(idx_shape)


# speedup vs baseline: 1.1943x; 1.1943x over previous
import functools

import jax
import jax.numpy as jnp
from jax import lax
from jax.experimental import pallas as pl
from jax.experimental.pallas import tpu as pltpu
from jax.experimental.pallas import tpu_sc as plsc

D = 256
K = 8192
N = 16 * 576

BN = 768
BK = 2048
NN = N // BN
NK = K // BK


def _argmin_body(x_ref, cb_ref, out_ref, best_val, best_idx):
    k = pl.program_id(1)

    @pl.when(k == 0)
    def _init():
        best_val[...] = jnp.full_like(best_val, jnp.inf)
        best_idx[...] = jnp.zeros_like(best_idx)

    x = x_ref[...]
    cb = cb_ref[...]
    a2 = jnp.sum(x * x, axis=1, keepdims=True)
    b2 = jnp.sum(cb * cb, axis=1)[None, :]
    s = lax.dot_general(x, cb, (((1,), (1,)), ((), ())),
                        preferred_element_type=jnp.float32)
    v = a2 + b2 - 2.0 * s

    m2 = jnp.min(v, axis=1, keepdims=True)
    m2c = jnp.maximum(m2, 0.0)
    sk = jnp.sqrt(m2c)
    s_next = lax.bitcast_convert_type(
        lax.bitcast_convert_type(sk, jnp.int32) + 1, jnp.float32)
    pb = lax.bitcast_convert_type(sk * s_next, jnp.int32)
    t = m2c
    for db in (-2, -1, 0, 1, 2):
        c = lax.bitcast_convert_type(jnp.maximum(pb + db, 0), jnp.float32)
        t = jnp.where((jnp.sqrt(c) == sk) & (c > t), c, t)

    cols = lax.broadcasted_iota(jnp.int32, v.shape, 1)
    li = jnp.min(jnp.where(v <= t, cols, jnp.int32(2**30)),
                 axis=1, keepdims=True)
    gi = li + k * BK

    upd = sk < best_val[...]
    best_idx[...] = jnp.where(upd, gi, best_idx[...])
    best_val[...] = jnp.where(upd, sk, best_val[...])

    @pl.when(k == pl.num_programs(1) - 1)
    def _flush():
        out_ref[0, 0, :] = best_idx[...][:, 0]


_argmin_call = pl.pallas_call(
    _argmin_body,
    grid=(NN, NK),
    in_specs=[
        pl.BlockSpec((BN, D), lambda n, k: (n, 0)),
        pl.BlockSpec((BK, D), lambda n, k: (k, 0)),
    ],
    out_specs=pl.BlockSpec((1, 1, BN), lambda n, k: (n, 0, 0)),
    out_shape=jax.ShapeDtypeStruct((NN, 1, BN), jnp.int32),
    scratch_shapes=[
        pltpu.VMEM((BN, 1), jnp.float32),
        pltpu.VMEM((BN, 1), jnp.int32),
    ],
    compiler_params=pltpu.CompilerParams(
        dimension_semantics=("parallel", "arbitrary"),
    ),
)


_NW = 32
_BPW = N // _NW


def _gather_body(table_hbm, idx_hbm, out_hbm, idx_v, rows_v, sem):
    wid = lax.axis_index("s") * 2 + lax.axis_index("c")
    base = wid * _BPW
    pltpu.sync_copy(idx_hbm.at[pl.ds(base, _BPW)], idx_v)
    pltpu.async_copy(table_hbm.at[idx_v], rows_v, sem).wait()
    pltpu.sync_copy(rows_v, out_hbm.at[pl.ds(base, _BPW)])


@functools.cache
def _gather_call():
    return functools.partial(
        pl.kernel,
        out_type=jax.ShapeDtypeStruct((N, D), jnp.float32),
        mesh=plsc.VectorSubcoreMesh(core_axis_name="c", subcore_axis_name="s"),
        scratch_types=[
            pltpu.VMEM((_BPW,), jnp.int32),
            pltpu.VMEM((_BPW, D), jnp.float32),
            pltpu.SemaphoreType.DMA,
        ],
    )(_gather_body)


def kernel(x, codebook):
    xf = x.reshape(N, D)
    idx_blocks = _argmin_call(xf, codebook)
    indices = idx_blocks.reshape(N)
    codes = _gather_call()(codebook, indices)
    idx_shape = list(x.shape)
    idx_shape[-1] = 1
    return codes.reshape(x.shape), indices.reshape(idx_shape)


# speedup vs baseline: 1.2137x; 1.0163x over previous
import functools

import jax
import jax.numpy as jnp
from jax import lax
from jax.experimental import pallas as pl
from jax.experimental.pallas import tpu as pltpu
from jax.experimental.pallas import tpu_sc as plsc

D = 256
K = 8192
N = 16 * 576

BN = 768
BK = 2048
NN = N // BN
NK = K // BK


def _argmin_body(x_ref, cb_ref, out_ref, best_val, best_idx, a2_s, b2_s, colf_s):
    n = pl.program_id(0)
    k = pl.program_id(1)

    @pl.when((n == 0) & (k == 0))
    def _cols():
        colf_s[...] = lax.broadcasted_iota(jnp.int32, (1, BK), 1).astype(
            jnp.float32)

    @pl.when(k == 0)
    def _init():
        best_val[...] = jnp.full_like(best_val, jnp.inf)
        best_idx[...] = jnp.zeros_like(best_idx)

    x = x_ref[...]
    cb = cb_ref[...]

    @pl.when(k == 0)
    def _a2():
        a2_s[...] = jnp.sum(x * x, axis=1, keepdims=True)

    @pl.when(n == 0)
    def _b2():
        b2_s[0, pl.ds(k * BK, BK)] = jnp.sum(cb * cb, axis=1)

    a2 = a2_s[...]
    b2 = b2_s[0, pl.ds(k * BK, BK)][None, :]
    s = lax.dot_general(x, cb, (((1,), (1,)), ((), ())),
                        preferred_element_type=jnp.float32)
    v = a2 + b2 - 2.0 * s

    m2 = jnp.min(v, axis=1, keepdims=True)
    m2c = jnp.maximum(m2, 0.0)
    sk = jnp.sqrt(m2c)
    s_next = lax.bitcast_convert_type(
        lax.bitcast_convert_type(sk, jnp.int32) + 1, jnp.float32)
    pb = lax.bitcast_convert_type(sk * s_next, jnp.int32)
    t = m2c
    for db in (-2, -1, 0, 1, 2):
        c = lax.bitcast_convert_type(jnp.maximum(pb + db, 0), jnp.float32)
        t = jnp.where((jnp.sqrt(c) == sk) & (c > t), c, t)

    cols = colf_s[...]
    li_f = jnp.min(jnp.where(v <= t, cols, jnp.float32(jnp.inf)),
                   axis=1, keepdims=True)
    gi = li_f.astype(jnp.int32) + k * BK

    upd = sk < best_val[...]
    best_idx[...] = jnp.where(upd, gi, best_idx[...])
    best_val[...] = jnp.where(upd, sk, best_val[...])

    @pl.when(k == pl.num_programs(1) - 1)
    def _flush():
        out_ref[0, 0, :] = best_idx[...][:, 0]


_argmin_call = pl.pallas_call(
    _argmin_body,
    grid=(NN, NK),
    in_specs=[
        pl.BlockSpec((BN, D), lambda n, k: (n, 0)),
        pl.BlockSpec((BK, D), lambda n, k: (k, 0)),
    ],
    out_specs=pl.BlockSpec((1, 1, BN), lambda n, k: (n, 0, 0)),
    out_shape=jax.ShapeDtypeStruct((NN, 1, BN), jnp.int32),
    scratch_shapes=[
        pltpu.VMEM((BN, 1), jnp.float32),
        pltpu.VMEM((BN, 1), jnp.int32),
        pltpu.VMEM((BN, 1), jnp.float32),
        pltpu.VMEM((1, K), jnp.float32),
        pltpu.VMEM((1, BK), jnp.float32),
    ],
    compiler_params=pltpu.CompilerParams(
        dimension_semantics=("parallel", "arbitrary"),
    ),
)


_NW = 32
_BPW = N // _NW


def _gather_body(table_hbm, idx_hbm, out_hbm, idx_v, rows_v, sem):
    wid = lax.axis_index("s") * 2 + lax.axis_index("c")
    base = wid * _BPW
    pltpu.sync_copy(idx_hbm.at[pl.ds(base, _BPW)], idx_v)
    pltpu.async_copy(table_hbm.at[idx_v], rows_v, sem).wait()
    pltpu.sync_copy(rows_v, out_hbm.at[pl.ds(base, _BPW)])


@functools.cache
def _gather_call():
    return functools.partial(
        pl.kernel,
        out_type=jax.ShapeDtypeStruct((N, D), jnp.float32),
        mesh=plsc.VectorSubcoreMesh(core_axis_name="c", subcore_axis_name="s"),
        scratch_types=[
            pltpu.VMEM((_BPW,), jnp.int32),
            pltpu.VMEM((_BPW, D), jnp.float32),
            pltpu.SemaphoreType.DMA,
        ],
    )(_gather_body)


def kernel(x, codebook):
    xf = x.reshape(N, D)
    idx_blocks = _argmin_call(xf, codebook)
    indices = idx_blocks.reshape(N)
    codes = _gather_call()(codebook, indices)
    idx_shape = list(x.shape)
    idx_shape[-1] = 1
    return codes.reshape(x.shape), indices.reshape(idx_shape)


# speedup vs baseline: 1.6319x; 1.3446x over previous
import functools

import jax
import jax.numpy as jnp
from jax import lax
from jax.experimental import pallas as pl
from jax.experimental.pallas import tpu as pltpu
from jax.experimental.pallas import tpu_sc as plsc

D = 256
K = 8192
N = 16 * 576

BN = 512
NN = N // BN
CH = 2048
NCH = K // CH


def _argmin_body(x_ref, cb_ref, out_ref, v_s, b2_s, colf_s):
    n = pl.program_id(0)

    @pl.when(n == 0)
    def _prologue():
        cb = cb_ref[...]
        b2_s[...] = jnp.sum(cb * cb, axis=1)[None, :]
        colf_s[...] = lax.broadcasted_iota(jnp.int32, (1, K), 1).astype(
            jnp.float32)

    x2 = x_ref[...] * 2.0
    a2h = jnp.sum(x_ref[...] * x_ref[...], axis=1, keepdims=True)

    m2 = jnp.full((BN, 1), jnp.inf, dtype=jnp.float32)
    for j in range(NCH):
        cbj = cb_ref[pl.ds(j * CH, CH), :]
        s2 = lax.dot_general(x2, cbj, (((1,), (1,)), ((), ())),
                             preferred_element_type=jnp.float32)
        b2j = b2_s[0, pl.ds(j * CH, CH)][None, :]
        vj = a2h + b2j - s2
        v_s[:, pl.ds(j * CH, CH)] = vj
        m2 = jnp.minimum(m2, jnp.min(vj, axis=1, keepdims=True))

    m2c = jnp.maximum(m2, 0.0)
    sk = jnp.sqrt(m2c)
    s_next = lax.bitcast_convert_type(
        lax.bitcast_convert_type(sk, jnp.int32) + 1, jnp.float32)
    pb = lax.bitcast_convert_type(sk * s_next, jnp.int32)
    t = m2c
    for db in (-2, -1, 0, 1, 2):
        c = lax.bitcast_convert_type(jnp.maximum(pb + db, 0), jnp.float32)
        t = jnp.where((jnp.sqrt(c) == sk) & (c > t), c, t)

    li = jnp.full((BN, 1), jnp.inf, dtype=jnp.float32)
    for j in range(NCH):
        vj = v_s[:, pl.ds(j * CH, CH)]
        colj = colf_s[0, pl.ds(j * CH, CH)][None, :]
        lij = jnp.min(jnp.where(vj <= t, colj, jnp.float32(jnp.inf)),
                      axis=1, keepdims=True)
        li = jnp.minimum(li, lij)

    out_ref[0, 0, :] = li.astype(jnp.int32)[:, 0]


_argmin_call = pl.pallas_call(
    _argmin_body,
    grid=(NN,),
    in_specs=[
        pl.BlockSpec((BN, D), lambda n: (n, 0)),
        pl.BlockSpec((K, D), lambda n: (0, 0)),
    ],
    out_specs=pl.BlockSpec((1, 1, BN), lambda n: (n, 0, 0)),
    out_shape=jax.ShapeDtypeStruct((NN, 1, BN), jnp.int32),
    scratch_shapes=[
        pltpu.VMEM((BN, K), jnp.float32),
        pltpu.VMEM((1, K), jnp.float32),
        pltpu.VMEM((1, K), jnp.float32),
    ],
    compiler_params=pltpu.CompilerParams(
        dimension_semantics=("arbitrary",),
    ),
)


_NW = 32
_BPW = N // _NW


def _gather_body(table_hbm, idx_hbm, out_hbm, idx_v, rows_v, sem):
    wid = lax.axis_index("s") * 2 + lax.axis_index("c")
    base = wid * _BPW
    pltpu.sync_copy(idx_hbm.at[pl.ds(base, _BPW)], idx_v)
    pltpu.async_copy(table_hbm.at[idx_v], rows_v, sem).wait()
    pltpu.sync_copy(rows_v, out_hbm.at[pl.ds(base, _BPW)])


@functools.cache
def _gather_call():
    return functools.partial(
        pl.kernel,
        out_type=jax.ShapeDtypeStruct((N, D), jnp.float32),
        mesh=plsc.VectorSubcoreMesh(core_axis_name="c", subcore_axis_name="s"),
        scratch_types=[
            pltpu.VMEM((_BPW,), jnp.int32),
            pltpu.VMEM((_BPW, D), jnp.float32),
            pltpu.SemaphoreType.DMA,
        ],
    )(_gather_body)


def kernel(x, codebook):
    xf = x.reshape(N, D)
    idx_blocks = _argmin_call(xf, codebook)
    indices = idx_blocks.reshape(N)
    codes = _gather_call()(codebook, indices)
    idx_shape = list(x.shape)
    idx_shape[-1] = 1
    return codes.reshape(x.shape), indices.reshape(idx_shape)
